# baseline (speedup 1.0000x reference)
import functools

import jax
import jax.numpy as jnp
from jax import lax
from jax.experimental import pallas as pl
from jax.experimental.pallas import tpu as pltpu
from jax.experimental.pallas import tpu_sc as plsc

VOCAB = 1_000_000
D = 64
BATCH = 4096
SEQ = 50
B = BATCH * SEQ

NC = 2
NS = 16
NW = NC * NS
BL = BATCH // NW
B_PER_W = BL * SEQ
CHUNK = BL
NROT = 2
TR = D // 8

_mesh = plsc.VectorSubcoreMesh(
    core_axis_name="c", subcore_axis_name="s", num_cores=NC, num_subcores=NS
)


@functools.partial(
    pl.kernel,
    out_type=jax.ShapeDtypeStruct((SEQ, TR, NW, 8, 128), jnp.float32),
    mesh=_mesh,
    scratch_types=[
        pltpu.VMEM((B_PER_W,), jnp.int32),
        pltpu.VMEM((B_PER_W,), jnp.int32),
        pltpu.VMEM((NROT, CHUNK, D), jnp.float32),
        pltpu.VMEM((TR, 8, 128), jnp.float32),
        pltpu.SemaphoreType.DMA((NROT,)),
        pltpu.SemaphoreType.DMA((TR,)),
    ],
    compiler_params=pltpu.CompilerParams(
        use_tc_tiling_on_sc=False, needs_layout_passes=False
    ),
)
def _gather_kernel(idx_hbm, table_hbm, out_hbm, idx_bs, idx_sb, rows_v, obuf,
                   gsem, osem):
    wid = lax.axis_index("s") * NC + lax.axis_index("c")
    base = wid * B_PER_W

    pltpu.sync_copy(idx_hbm.at[pl.ds(base, B_PER_W)], idx_bs)

    iota = lax.iota(jnp.int32, 16)

    def reorder(s, _):
        for lg in range(BL // 16):
            src = (lg * 16 + iota) * SEQ + s
            v = plsc.load_gather(idx_bs, [src])
            idx_sb[pl.ds(s * BL + lg * 16, 16)] = v
        return _

    lax.fori_loop(0, SEQ, reorder, None, unroll=False)

    def gather(c, slot):
        return pltpu.make_async_copy(
            table_hbm.at[idx_sb.at[pl.ds(c * CHUNK, CHUNK)]],
            rows_v.at[slot],
            gsem.at[slot],
        )

    def write(c, tr):
        return pltpu.make_async_copy(
            obuf.at[tr],
            out_hbm.at[c, tr, wid],
            osem.at[tr],
        )

    def shuffle(slot, tr):
        for r in range(8):
            col = jnp.full((16,), 8 * tr + r, jnp.int32)
            for lg in range(BL // 16):
                row = lg * 16 + iota
                v = plsc.load_gather(rows_v.at[slot], [row, col])
                obuf[tr, r, pl.ds(lg * 16, 16)] = v

    def step(c, slot, first=False, issue=True):
        if issue:
            gather(c + 1, 1 - slot).start()
        gather(c, slot).wait()
        for tr in range(TR):
            if not first:
                write(c - 1, tr).wait()
            shuffle(slot, tr)
            write(c, tr).start()

    gather(0, 0).start()
    step(0, 0, first=True)

    def group(p, _):
        c = 1 + 2 * p
        step(c, 1)
        step(c + 1, 0)
        return _

    lax.fori_loop(0, (SEQ - 2) // 2, group, None, unroll=False)

    step(SEQ - 1, 1, issue=False)

    for tr in range(TR):
        write(SEQ - 1, tr).wait()


def kernel(inputs, embedding_table):
    flat_idx = inputs.reshape(B)
    out5 = _gather_kernel(flat_idx, embedding_table)
    return out5.transpose(2, 4, 0, 1, 3).reshape(BATCH, SEQ, D)


# speedup vs baseline: 1.2196x; 1.2196x over previous
import functools

import jax
import jax.numpy as jnp
from jax import lax
from jax.experimental import pallas as pl
from jax.experimental.pallas import tpu as pltpu
from jax.experimental.pallas import tpu_sc as plsc

VOCAB = 1_000_000
D = 64
BATCH = 4096
SEQ = 50
B = BATCH * SEQ

NC = 2
NS = 16
NW = NC * NS
B_PER_W = B // NW
CHUNK = 128
NCHUNK = B_PER_W // CHUNK
NBUF = 5
LOOKAHEAD = NBUF - 1

_mesh = plsc.VectorSubcoreMesh(
    core_axis_name="c", subcore_axis_name="s", num_cores=NC, num_subcores=NS
)


@functools.partial(
    pl.kernel,
    out_type=jax.ShapeDtypeStruct((B, D), jnp.float32),
    mesh=_mesh,
    scratch_types=[
        pltpu.VMEM((B_PER_W,), jnp.int32),
        pltpu.VMEM((NBUF, CHUNK, D), jnp.float32),
        pltpu.SemaphoreType.DMA((NBUF,)),
        pltpu.SemaphoreType.DMA((NBUF,)),
    ],
    compiler_params=pltpu.CompilerParams(use_tc_tiling_on_sc=False),
)
def _gather_kernel(idx_hbm, table_hbm, out_hbm, idx_v, rows_v, gsem, osem):
    wid = lax.axis_index("s") * NC + lax.axis_index("c")
    base = wid * B_PER_W

    pltpu.sync_copy(idx_hbm.at[pl.ds(base, B_PER_W)], idx_v)

    def gather(c, slot):
        return pltpu.make_async_copy(
            table_hbm.at[idx_v.at[pl.ds(c * CHUNK, CHUNK)]],
            rows_v.at[slot],
            gsem.at[slot],
        )

    def write(c, slot):
        return pltpu.make_async_copy(
            rows_v.at[slot],
            out_hbm.at[pl.ds(base + c * CHUNK, CHUNK)],
            osem.at[slot],
        )

    for c in range(LOOKAHEAD):
        gather(c, c).start()

    def step(c, b, first=False, issue=True):
        if issue:
            g_slot = (b + LOOKAHEAD) % NBUF
            if not first:
                write(c + LOOKAHEAD - NBUF, g_slot).wait()
            gather(c + LOOKAHEAD, g_slot).start()
        gather(c, b).wait()
        write(c, b).start()

    for b in range(NBUF):
        step(b, b, first=(b == 0))

    def group(p, _):
        for b in range(NBUF):
            step(p * NBUF + b, b)
        return _

    lax.fori_loop(1, NCHUNK // NBUF - 1, group, None, unroll=False)

    last = NCHUNK - NBUF
    for b in range(NBUF):
        step(last + b, b, issue=(last + b + LOOKAHEAD < NCHUNK))

    for c in range(NCHUNK - NBUF, NCHUNK):
        write(c, c % NBUF).wait()


def kernel(inputs, embedding_table):
    flat_idx = inputs.reshape(B)
    out = _gather_kernel(flat_idx, embedding_table)
    return out.reshape(BATCH, SEQ, D)
